# baseline (speedup 1.0000x reference)
import functools

import jax
import jax.numpy as jnp
from jax import lax
from jax.experimental import pallas as pl
from jax.experimental.pallas import tpu as pltpu
from jax.experimental.pallas import tpu_sc as plsc

B = 16384
K = 16
L = 16
NC = 2
NS = 16
NW = NC * NS
BPW = B // NW
NJ = BPW // L


def _body(gu_hbm, gi_hbm, users_hbm, items_hbm, bu_hbm, bi_hbm, mu_hbm,
          out_hbm, gu_v, gi_v, iu_v, ii_v, bu_v, bi_v, mu_v, out_v, sem):
    wid = lax.axis_index("s") * NC + lax.axis_index("c")
    base = wid * BPW

    pltpu.sync_copy(users_hbm.at[pl.ds(base, BPW)], iu_v)
    pltpu.sync_copy(items_hbm.at[pl.ds(base, BPW)], ii_v)

    cp_u = pltpu.async_copy(bu_hbm.at[iu_v], bu_v, sem)
    cp_i = pltpu.async_copy(bi_hbm.at[ii_v], bi_v, sem)
    pltpu.sync_copy(gu_hbm.at[pl.ds(base * K, BPW * K)], gu_v)
    pltpu.sync_copy(gi_hbm.at[pl.ds(base * K, BPW * K)], gi_v)
    pltpu.sync_copy(mu_hbm, mu_v)
    cp_u.wait()
    cp_i.wait()

    mu = mu_v[...]
    lane16 = lax.iota(jnp.int32, L) * K

    def group(j, carry):
        gbase = j * (L * K) + lane16
        acc = jnp.zeros((L,), jnp.float32)
        for k in range(K):
            idx = gbase + k
            a = plsc.load_gather(gu_v, [idx])
            b = plsc.load_gather(gi_v, [idx])
            acc = acc + a * b
        bu = bu_v[pl.ds(j * L, L)]
        bi = bi_v[pl.ds(j * L, L)]
        out_v[pl.ds(j * L, L)] = acc + bu + bi + mu
        return carry

    lax.fori_loop(0, NJ, group, 0)

    pltpu.sync_copy(out_v, out_hbm.at[pl.ds(base, BPW)])


@jax.jit
def _run(gu, gi, users, items, bu_flat, bi_flat, mu_b):
    mesh = plsc.VectorSubcoreMesh(core_axis_name="c", subcore_axis_name="s")
    f = functools.partial(
        pl.kernel,
        mesh=mesh,
        out_type=jax.ShapeDtypeStruct((B,), jnp.float32),
        scratch_types=[
            pltpu.VMEM((BPW * K,), jnp.float32),
            pltpu.VMEM((BPW * K,), jnp.float32),
            pltpu.VMEM((BPW,), jnp.int32),
            pltpu.VMEM((BPW,), jnp.int32),
            pltpu.VMEM((BPW,), jnp.float32),
            pltpu.VMEM((BPW,), jnp.float32),
            pltpu.VMEM((L,), jnp.float32),
            pltpu.VMEM((BPW,), jnp.float32),
            pltpu.SemaphoreType.DMA,
        ],
        compiler_params=pltpu.CompilerParams(needs_layout_passes=False,
                                             use_tc_tiling_on_sc=False),
    )(_body)
    return f(gu, gi, users, items, bu_flat, bi_flat, mu_b)


def kernel(gu, gi, users, items, Bu, Bi, Mu):
    mu_b = jnp.broadcast_to(Mu.reshape(()), (L,))
    return _run(gu.reshape(-1), gi.reshape(-1),
                users.astype(jnp.int32), items.astype(jnp.int32),
                Bu.reshape(-1), Bi.reshape(-1), mu_b)


# speedup vs baseline: 1.0689x; 1.0689x over previous
import functools

import jax
import jax.numpy as jnp
from jax import lax
from jax.experimental import pallas as pl
from jax.experimental.pallas import tpu as pltpu
from jax.experimental.pallas import tpu_sc as plsc

B = 16384
K = 16
L = 16
NC = 2
NS = 16
NW = NC * NS
BPW = B // NW
NCH = 4
CHUNK = BPW // NCH
NJC = CHUNK // L


def _body(gu_hbm, gi_hbm, users_hbm, items_hbm, bu_hbm, bi_hbm, mu_hbm,
          out_hbm, gu0, gi0, gu1, gi1, iu_v, ii_v, bu_v, bi_v, mu_v, out_v,
          semb, sem0, sem1):
    wid = lax.axis_index("s") * NC + lax.axis_index("c")
    base = wid * BPW

    pltpu.sync_copy(users_hbm.at[pl.ds(base, BPW)], iu_v)
    pltpu.sync_copy(items_hbm.at[pl.ds(base, BPW)], ii_v)
    cp_u = pltpu.async_copy(bu_hbm.at[iu_v], bu_v, semb)
    cp_i = pltpu.async_copy(bi_hbm.at[ii_v], bi_v, semb)

    gbufs = (gu0, gu1)
    ibufs = (gi0, gi1)
    sems = (sem0, sem1)

    def start(ch):
        cbase = base + ch * CHUNK
        s = sems[ch % 2]
        return (pltpu.async_copy(gu_hbm.at[pl.ds(cbase, CHUNK), :],
                                 gbufs[ch % 2], s),
                pltpu.async_copy(gi_hbm.at[pl.ds(cbase, CHUNK), :],
                                 ibufs[ch % 2], s))

    pend = start(0)
    pltpu.sync_copy(mu_hbm, mu_v)
    cp_u.wait()
    cp_i.wait()

    mu = mu_v[...]
    lane = lax.iota(jnp.int32, L)

    for ch in range(NCH):
        nxt = start(ch + 1) if ch + 1 < NCH else None
        pend[0].wait()
        pend[1].wait()
        gu_v = gbufs[ch % 2]
        gi_v = ibufs[ch % 2]

        def group(j, carry, gu_v=gu_v, gi_v=gi_v, ch=ch):
            rows = j * L + lane
            acc = jnp.zeros((L,), jnp.float32)
            for k in range(K):
                cols = jnp.full((L,), k, jnp.int32)
                a = plsc.load_gather(gu_v, [rows, cols])
                b = plsc.load_gather(gi_v, [rows, cols])
                acc = acc + a * b
            off = ch * CHUNK + j * L
            bu = bu_v[pl.ds(off, L)]
            bi = bi_v[pl.ds(off, L)]
            out_v[pl.ds(off, L)] = acc + bu + bi + mu
            return carry

        lax.fori_loop(0, NJC, group, 0)
        pend = nxt

    pltpu.sync_copy(out_v, out_hbm.at[pl.ds(base, BPW)])


@jax.jit
def _run(gu, gi, users, items, bu_flat, bi_flat, mu_b):
    mesh = plsc.VectorSubcoreMesh(core_axis_name="c", subcore_axis_name="s")
    f = functools.partial(
        pl.kernel,
        mesh=mesh,
        out_type=jax.ShapeDtypeStruct((B,), jnp.float32),
        scratch_types=[
            pltpu.VMEM((CHUNK, K), jnp.float32),
            pltpu.VMEM((CHUNK, K), jnp.float32),
            pltpu.VMEM((CHUNK, K), jnp.float32),
            pltpu.VMEM((CHUNK, K), jnp.float32),
            pltpu.VMEM((BPW,), jnp.int32),
            pltpu.VMEM((BPW,), jnp.int32),
            pltpu.VMEM((BPW,), jnp.float32),
            pltpu.VMEM((BPW,), jnp.float32),
            pltpu.VMEM((L,), jnp.float32),
            pltpu.VMEM((BPW,), jnp.float32),
            pltpu.SemaphoreType.DMA,
            pltpu.SemaphoreType.DMA,
            pltpu.SemaphoreType.DMA,
        ],
        compiler_params=pltpu.CompilerParams(needs_layout_passes=False),
    )(_body)
    return f(gu, gi, users, items, bu_flat, bi_flat, mu_b)


def kernel(gu, gi, users, items, Bu, Bi, Mu):
    mu_b = jnp.broadcast_to(Mu.reshape(()), (L,))
    return _run(gu, gi, users.astype(jnp.int32), items.astype(jnp.int32),
                Bu.reshape(-1), Bi.reshape(-1), mu_b)
